# baseline (speedup 1.0000x reference)
import functools

import jax
import jax.numpy as jnp
from jax import lax
from jax.experimental import pallas as pl
from jax.experimental.pallas import tpu as pltpu
from jax.experimental.pallas import tpu_sc as plsc

_M = 1000000
_C = float((1.0 / _M) * (1.0 - 2.0**-22))

_N = 3276800


def _mod_vec(x):
    q = (x.astype(jnp.float32) * jnp.float32(_C)).astype(jnp.int32)
    r = x - q * _M
    ru = lax.bitcast_convert_type(r, jnp.uint32)
    su = lax.bitcast_convert_type(r - _M, jnp.uint32)
    return lax.bitcast_convert_type(jnp.minimum(ru, su), jnp.int32)


_NC, _NS, _L = 2, 16, 16
_NW = _NC * _NS
_PER_W = _N // _NW
_CHUNK = 25600
_NCHUNK = _PER_W // _CHUNK


def _sc_body(src_hbm, dst_hbm, in0, in1, ou0, ou1, ls0, ls1, ss0, ss1):
    wid = lax.axis_index("s") * _NC + lax.axis_index("c")
    base = wid * _PER_W

    inbufs = (in0, in1)
    outbufs = (ou0, ou1)
    lsems = (ls0, ls1)
    ssems = (ss0, ss1)

    def start_load(i):
        return pltpu.async_copy(
            src_hbm.at[pl.ds(base + i * _CHUNK, _CHUNK)],
            inbufs[i % 2], lsems[i % 2])

    def start_store(i):
        return pltpu.async_copy(
            outbufs[i % 2],
            dst_hbm.at[pl.ds(base + i * _CHUNK, _CHUNK)], ssems[i % 2])

    loads = {0: start_load(0), 1: start_load(1)}
    stores = {}
    for i in range(_NCHUNK):
        loads.pop(i).wait()
        if i >= 2:
            stores.pop(i - 2).wait()
        inb = inbufs[i % 2]
        outb = outbufs[i % 2]

        @plsc.parallel_loop(0, _CHUNK, _L, unroll=4)
        def _(v, inb=inb, outb=outb):
            outb[pl.ds(v, _L)] = _mod_vec(inb[pl.ds(v, _L)])

        stores[i] = start_store(i)
        if i + 2 < _NCHUNK:
            loads[i + 2] = start_load(i + 2)
    stores.pop(_NCHUNK - 2).wait()
    stores.pop(_NCHUNK - 1).wait()


_sc_mod = functools.partial(
    pl.kernel,
    out_type=jax.ShapeDtypeStruct((_N,), jnp.int32),
    mesh=plsc.VectorSubcoreMesh(core_axis_name="c", subcore_axis_name="s"),
    scratch_types=(
        pltpu.VMEM((_CHUNK,), jnp.int32),
        pltpu.VMEM((_CHUNK,), jnp.int32),
        pltpu.VMEM((_CHUNK,), jnp.int32),
        pltpu.VMEM((_CHUNK,), jnp.int32),
        pltpu.SemaphoreType.DMA,
        pltpu.SemaphoreType.DMA,
        pltpu.SemaphoreType.DMA,
        pltpu.SemaphoreType.DMA,
    ),
)(_sc_body)


_TC_BLK = 409600


def _mod_vec_tc(x):
    q = (x.astype(jnp.float32) * jnp.float32(_C)).astype(jnp.int32)
    r = x - q * _M
    return jnp.where(r >= _M, r - _M, r)


def _tc_body(vals_ref, vout_ref):
    vout_ref[...] = _mod_vec_tc(vals_ref[...])


_tc_mod = pl.pallas_call(
    _tc_body,
    grid=(_N // _TC_BLK,),
    in_specs=[pl.BlockSpec((_TC_BLK,), lambda i: (i,))],
    out_specs=pl.BlockSpec((_TC_BLK,), lambda i: (i,)),
    out_shape=jax.ShapeDtypeStruct((_N,), jnp.int32),
)


def kernel(id_list_offsets, id_list_values, id_score_list_offsets,
           id_score_list_keys, id_score_list_scores):
    idx_keys = _sc_mod(id_score_list_keys)
    vout = _tc_mod(id_list_values)
    return (id_list_offsets, vout, id_score_list_offsets, idx_keys,
            id_score_list_scores.astype(jnp.float32))


# speedup vs baseline: 1.1484x; 1.1484x over previous
import functools

import jax
import jax.numpy as jnp
from jax import lax
from jax.experimental import pallas as pl
from jax.experimental.pallas import tpu as pltpu
from jax.experimental.pallas import tpu_sc as plsc

_M = 1000000
_C = float((1.0 / _M) * (1.0 - 2.0**-22))

_N = 3276800


def _mod_vec(x):
    q = (x.astype(jnp.float32) * jnp.float32(_C)).astype(jnp.int32)
    r = x - q * _M
    ru = lax.bitcast_convert_type(r, jnp.uint32)
    su = lax.bitcast_convert_type(r - _M, jnp.uint32)
    return lax.bitcast_convert_type(jnp.minimum(ru, su), jnp.int32)


_NC, _NS, _L = 2, 16, 16
_NW = _NC * _NS
_PER_W = _N // _NW
_CHUNK = 25600
_NCHUNK = _PER_W // _CHUNK


def _sc_body(src_hbm, dst_hbm, in0, in1, ou0, ou1, ls0, ls1, ss0, ss1):
    wid = lax.axis_index("s") * _NC + lax.axis_index("c")
    base = wid * _PER_W

    inbufs = (in0, in1)
    outbufs = (ou0, ou1)
    lsems = (ls0, ls1)
    ssems = (ss0, ss1)

    def start_load(i):
        return pltpu.async_copy(
            src_hbm.at[pl.ds(base + i * _CHUNK, _CHUNK)],
            inbufs[i % 2], lsems[i % 2])

    def start_store(i):
        return pltpu.async_copy(
            outbufs[i % 2],
            dst_hbm.at[pl.ds(base + i * _CHUNK, _CHUNK)], ssems[i % 2])

    loads = {0: start_load(0), 1: start_load(1)}
    stores = {}
    for i in range(_NCHUNK):
        loads.pop(i).wait()
        if i >= 2:
            stores.pop(i - 2).wait()
        inb = inbufs[i % 2]
        outb = outbufs[i % 2]

        @plsc.parallel_loop(0, _CHUNK, _L, unroll=8)
        def _(v, inb=inb, outb=outb):
            outb[pl.ds(v, _L)] = _mod_vec(inb[pl.ds(v, _L)])

        stores[i] = start_store(i)
        if i + 2 < _NCHUNK:
            loads[i + 2] = start_load(i + 2)
    stores.pop(_NCHUNK - 2).wait()
    stores.pop(_NCHUNK - 1).wait()


_sc_mod = functools.partial(
    pl.kernel,
    out_type=jax.ShapeDtypeStruct((_N,), jnp.int32),
    mesh=plsc.VectorSubcoreMesh(core_axis_name="c", subcore_axis_name="s"),
    scratch_types=(
        pltpu.VMEM((_CHUNK,), jnp.int32),
        pltpu.VMEM((_CHUNK,), jnp.int32),
        pltpu.VMEM((_CHUNK,), jnp.int32),
        pltpu.VMEM((_CHUNK,), jnp.int32),
        pltpu.SemaphoreType.DMA,
        pltpu.SemaphoreType.DMA,
        pltpu.SemaphoreType.DMA,
        pltpu.SemaphoreType.DMA,
    ),
)(_sc_body)


_TC_BLK = 409600


def _mod_vec_tc(x):
    q = (x.astype(jnp.float32) * jnp.float32(_C)).astype(jnp.int32)
    r = x - q * _M
    return jnp.where(r >= _M, r - _M, r)


def _tc_body(vals_ref, scores_ref, vout_ref, sout_ref):
    vout_ref[...] = _mod_vec_tc(vals_ref[...])
    sout_ref[...] = scores_ref[...]


_tc_mod_copy = pl.pallas_call(
    _tc_body,
    grid=(_N // _TC_BLK,),
    in_specs=[
        pl.BlockSpec((_TC_BLK,), lambda i: (i,)),
        pl.BlockSpec((_TC_BLK,), lambda i: (i,)),
    ],
    out_specs=[
        pl.BlockSpec((_TC_BLK,), lambda i: (i,)),
        pl.BlockSpec((_TC_BLK,), lambda i: (i,)),
    ],
    out_shape=[
        jax.ShapeDtypeStruct((_N,), jnp.int32),
        jax.ShapeDtypeStruct((_N,), jnp.float32),
    ],
)


def kernel(id_list_offsets, id_list_values, id_score_list_offsets,
           id_score_list_keys, id_score_list_scores):
    idx_keys = _sc_mod(id_score_list_keys)
    vout, sout = _tc_mod_copy(id_list_values, id_score_list_scores)
    return (id_list_offsets, vout, id_score_list_offsets, idx_keys, sout)


# speedup vs baseline: 1.1789x; 1.0266x over previous
import functools

import jax
import jax.numpy as jnp
from jax import lax
from jax.experimental import pallas as pl
from jax.experimental.pallas import tpu as pltpu
from jax.experimental.pallas import tpu_sc as plsc

_M = 1000000
_C = float((1.0 / _M) * (1.0 - 2.0**-22))

_N = 3276800


def _mod_vec(x):
    q = (x.astype(jnp.float32) * jnp.float32(_C)).astype(jnp.int32)
    r = x - q * _M
    ru = lax.bitcast_convert_type(r, jnp.uint32)
    su = lax.bitcast_convert_type(r - _M, jnp.uint32)
    return lax.bitcast_convert_type(jnp.minimum(ru, su), jnp.int32)


_NC, _NS, _L = 2, 16, 16
_NW = _NC * _NS
_PER_W = _N // _NW
_CHUNK = 25600
_NCHUNK = _PER_W // _CHUNK


def _sc_body(src_hbm, dst_hbm, in0, in1, ou0, ou1, ls0, ls1, ss0, ss1):
    wid = lax.axis_index("s") * _NC + lax.axis_index("c")
    base = wid * _PER_W

    inbufs = (in0, in1)
    outbufs = (ou0, ou1)
    lsems = (ls0, ls1)
    ssems = (ss0, ss1)

    def start_load(i):
        return pltpu.async_copy(
            src_hbm.at[pl.ds(base + i * _CHUNK, _CHUNK)],
            inbufs[i % 2], lsems[i % 2])

    def start_store(i):
        return pltpu.async_copy(
            outbufs[i % 2],
            dst_hbm.at[pl.ds(base + i * _CHUNK, _CHUNK)], ssems[i % 2])

    loads = {0: start_load(0), 1: start_load(1)}
    stores = {}
    for i in range(_NCHUNK):
        loads.pop(i).wait()
        if i >= 2:
            stores.pop(i - 2).wait()
        inb = inbufs[i % 2]
        outb = outbufs[i % 2]

        @plsc.parallel_loop(0, _CHUNK, _L, unroll=8)
        def _(v, inb=inb, outb=outb):
            outb[pl.ds(v, _L)] = _mod_vec(inb[pl.ds(v, _L)])

        stores[i] = start_store(i)
        if i + 2 < _NCHUNK:
            loads[i + 2] = start_load(i + 2)
    stores.pop(_NCHUNK - 2).wait()
    stores.pop(_NCHUNK - 1).wait()


_sc_mod = functools.partial(
    pl.kernel,
    out_type=jax.ShapeDtypeStruct((_N,), jnp.int32),
    mesh=plsc.VectorSubcoreMesh(core_axis_name="c", subcore_axis_name="s"),
    scratch_types=(
        pltpu.VMEM((_CHUNK,), jnp.int32),
        pltpu.VMEM((_CHUNK,), jnp.int32),
        pltpu.VMEM((_CHUNK,), jnp.int32),
        pltpu.VMEM((_CHUNK,), jnp.int32),
        pltpu.SemaphoreType.DMA,
        pltpu.SemaphoreType.DMA,
        pltpu.SemaphoreType.DMA,
        pltpu.SemaphoreType.DMA,
    ),
)(_sc_body)


_TC_BLK = 409600


def _mod_vec_tc(x):
    q = (x.astype(jnp.float32) * jnp.float32(_C)).astype(jnp.int32)
    r = x - q * _M
    return jnp.where(r >= _M, r - _M, r)


_NOFF = 16385


def _tc_body(vals_ref, scores_ref, off1_ref, off2_ref,
             vout_ref, sout_ref, o1out_ref, o2out_ref):
    vout_ref[...] = _mod_vec_tc(vals_ref[...])
    sout_ref[...] = scores_ref[...]

    @pl.when(pl.program_id(0) == 0)
    def _():
        o1out_ref[...] = off1_ref[...]
        o2out_ref[...] = off2_ref[...]


_tc_mod_copy = pl.pallas_call(
    _tc_body,
    grid=(_N // _TC_BLK,),
    in_specs=[
        pl.BlockSpec((_TC_BLK,), lambda i: (i,)),
        pl.BlockSpec((_TC_BLK,), lambda i: (i,)),
        pl.BlockSpec((_NOFF,), lambda i: (0,)),
        pl.BlockSpec((_NOFF,), lambda i: (0,)),
    ],
    out_specs=[
        pl.BlockSpec((_TC_BLK,), lambda i: (i,)),
        pl.BlockSpec((_TC_BLK,), lambda i: (i,)),
        pl.BlockSpec((_NOFF,), lambda i: (0,)),
        pl.BlockSpec((_NOFF,), lambda i: (0,)),
    ],
    out_shape=[
        jax.ShapeDtypeStruct((_N,), jnp.int32),
        jax.ShapeDtypeStruct((_N,), jnp.float32),
        jax.ShapeDtypeStruct((_NOFF,), jnp.int32),
        jax.ShapeDtypeStruct((_NOFF,), jnp.int32),
    ],
)


def kernel(id_list_offsets, id_list_values, id_score_list_offsets,
           id_score_list_keys, id_score_list_scores):
    idx_keys = _sc_mod(id_score_list_keys)
    vout, sout, off1, off2 = _tc_mod_copy(
        id_list_values, id_score_list_scores,
        id_list_offsets, id_score_list_offsets)
    return (off1, vout, off2, idx_keys, sout)
